# Initial kernel scaffold; baseline (speedup 1.0000x reference)
import jax
import jax.numpy as jnp
from jax.experimental import pallas as pl


def kernel(x, adj_vals, edge_index, W1, b1, W2, b2):
    raise NotImplementedError("write your pallas kernel here")


# speedup vs baseline: 3.7105x; 3.7105x over previous
import functools

import jax
import jax.numpy as jnp
from jax import lax
from jax.experimental import pallas as pl
from jax.experimental.pallas import tpu as pltpu
from jax.experimental.pallas import tpu_sc as plsc

_N = 10000
_E = 160000
_D_HALF = 128
_K = 128
_TILES = 16
_CHUNKS = 79
_EPAD = _TILES * _CHUNKS * _K
_BLK = 200
_NBLK = _N // _BLK
_MAXB = 4


def _spmm_sc(vals, src, dst, xa, xb):
    per_tile = _EPAD // _TILES
    mesh = plsc.VectorSubcoreMesh(core_axis_name="c", subcore_axis_name="s")

    @functools.partial(
        pl.kernel,
        mesh=mesh,
        out_type=(
            jax.ShapeDtypeStruct((_N, _D_HALF), jnp.float32),
            jax.ShapeDtypeStruct((_N, _D_HALF), jnp.float32),
        ),
        scratch_types=[
            pltpu.VMEM((_K,), jnp.int32),
            pltpu.VMEM((_K,), jnp.int32),
            pltpu.VMEM((_K,), jnp.float32),
            pltpu.VMEM((_K, _D_HALF), jnp.float32),
            pltpu.VMEM_SHARED((_N, _D_HALF), jnp.float32),
            pltpu.SemaphoreType.DMA,
        ],
    )
    def k(vals_h, src_h, dst_h, xa_h, xb_h, oa_h, ob_h,
          src_v, dst_v, val_v, rows_v, acc, sem):
        c = lax.axis_index("c")
        s = lax.axis_index("s")
        tile_base = s * per_tile

        zeros16 = jnp.zeros((16,), jnp.float32)

        def zrow(i, _):
            for j in range(_D_HALF // 16):
                rows_v[i, pl.ds(j * 16, 16)] = zeros16
            return _

        lax.fori_loop(0, _K, zrow, None)

        def zinit(i, _):
            blk = s + _TILES * i

            @pl.when(blk < _NBLK)
            def _z():
                base = blk * _BLK
                pltpu.sync_copy(rows_v, acc.at[pl.ds(base, _K)])
                pltpu.sync_copy(rows_v.at[pl.ds(0, _BLK - _K)],
                                acc.at[pl.ds(base + _K, _BLK - _K)])
            return _

        lax.fori_loop(0, _MAXB, zinit, None)
        plsc.subcore_barrier()

        def chunk(ci, _):
            base = tile_base + ci * _K
            pltpu.sync_copy(src_h.at[pl.ds(base, _K)], src_v)
            pltpu.sync_copy(dst_h.at[pl.ds(base, _K)], dst_v)
            pltpu.sync_copy(vals_h.at[pl.ds(base, _K)], val_v)

            @pl.when(c == 0)
            def _g0():
                pltpu.async_copy(xa_h.at[src_v], rows_v, sem).wait()

            @pl.when(c == 1)
            def _g1():
                pltpu.async_copy(xb_h.at[src_v], rows_v, sem).wait()

            def edge_group(g, _):
                vv = val_v[pl.ds(g * 16, 16)]
                for l in range(16):
                    v = vv[l]
                    e = g * 16 + l
                    for j in range(_D_HALF // 16):
                        sl = pl.ds(j * 16, 16)
                        rows_v[e, sl] = rows_v[e, sl] * v
                return _

            lax.fori_loop(0, _K // 16, edge_group, None)
            pltpu.sync_copy(rows_v, acc.at[dst_v], add=True)
            return _

        lax.fori_loop(0, _CHUNKS, chunk, None)
        plsc.subcore_barrier()

        def epi(i, _):
            blk = s + _TILES * i

            @pl.when(blk < _NBLK)
            def _e():
                sl = pl.ds(blk * _BLK, _BLK)

                @pl.when(c == 0)
                def _w0():
                    pltpu.sync_copy(acc.at[sl], oa_h.at[sl])

                @pl.when(c == 1)
                def _w1():
                    pltpu.sync_copy(acc.at[sl], ob_h.at[sl])
            return _

        lax.fori_loop(0, _MAXB, epi, None)

    return k(vals, src, dst, xa, xb)


def _dense_tc(sa, sb, W1a, W1b, b1r, W2a, W2b):
    bm = 1000

    def body(sa_r, sb_r, w1a_r, w1b_r, b1_r, w2a_r, w2b_r, ta_r, tb_r):
        h = jnp.dot(sa_r[...], w1a_r[...], preferred_element_type=jnp.float32)
        h = h + jnp.dot(sb_r[...], w1b_r[...], preferred_element_type=jnp.float32)
        h = jnp.maximum(h + b1_r[...], 0.0)
        ta_r[...] = jnp.dot(h, w2a_r[...], preferred_element_type=jnp.float32)
        tb_r[...] = jnp.dot(h, w2b_r[...], preferred_element_type=jnp.float32)

    hid = W1a.shape[1]
    return pl.pallas_call(
        body,
        grid=(_N // bm,),
        in_specs=[
            pl.BlockSpec((bm, _D_HALF), lambda i: (i, 0)),
            pl.BlockSpec((bm, _D_HALF), lambda i: (i, 0)),
            pl.BlockSpec((_D_HALF, hid), lambda i: (0, 0)),
            pl.BlockSpec((_D_HALF, hid), lambda i: (0, 0)),
            pl.BlockSpec((1, hid), lambda i: (0, 0)),
            pl.BlockSpec((hid, _D_HALF), lambda i: (0, 0)),
            pl.BlockSpec((hid, _D_HALF), lambda i: (0, 0)),
        ],
        out_specs=[
            pl.BlockSpec((bm, _D_HALF), lambda i: (i, 0)),
            pl.BlockSpec((bm, _D_HALF), lambda i: (i, 0)),
        ],
        out_shape=[
            jax.ShapeDtypeStruct((_N, _D_HALF), jnp.float32),
            jax.ShapeDtypeStruct((_N, _D_HALF), jnp.float32),
        ],
    )(sa, sb, W1a, W1b, b1r, W2a, W2b)


def kernel(x, adj_vals, edge_index, W1, b1, W2, b2):
    src = edge_index[0].astype(jnp.int32)
    dst = edge_index[1].astype(jnp.int32)
    pad = _EPAD - _E
    src = jnp.concatenate([src, jnp.zeros((pad,), jnp.int32)])
    dst = jnp.concatenate([dst, jnp.zeros((pad,), jnp.int32)])
    vals = jnp.concatenate([adj_vals, jnp.zeros((pad,), jnp.float32)])

    xa = x[:, :_D_HALF]
    xb = x[:, _D_HALF:]
    sa, sb = _spmm_sc(vals, src, dst, xa, xb)

    ta, tb = _dense_tc(sa, sb, W1[:_D_HALF], W1[_D_HALF:],
                       b1.reshape(1, -1), W2[:, :_D_HALF], W2[:, _D_HALF:])

    oa, ob = _spmm_sc(vals, src, dst, ta, tb)
    return jnp.concatenate([oa, ob], axis=1) + b2


# speedup vs baseline: 4.2628x; 1.1489x over previous
import functools

import jax
import jax.numpy as jnp
from jax import lax
from jax.experimental import pallas as pl
from jax.experimental.pallas import tpu as pltpu
from jax.experimental.pallas import tpu_sc as plsc

_N = 10000
_E = 160000
_D_HALF = 128
_K = 128
_TILES = 16
_CHUNKS = 80
_EPAD = _TILES * _CHUNKS * _K
_NBUF = 3
_BLK = 200
_NBLK = _N // _BLK
_MAXB = 4


def _spmm_sc(combo, vals, xa, xb):
    mesh = plsc.VectorSubcoreMesh(core_axis_name="c", subcore_axis_name="s")

    @functools.partial(
        pl.kernel,
        mesh=mesh,
        out_type=(
            jax.ShapeDtypeStruct((_N, _D_HALF), jnp.float32),
            jax.ShapeDtypeStruct((_N, _D_HALF), jnp.float32),
        ),
        scratch_types=[
            pltpu.VMEM((_NBUF, 2, _K), jnp.int32),
            pltpu.VMEM((_NBUF, _K), jnp.float32),
        ] + [pltpu.VMEM((_K, _D_HALF), jnp.float32)] * _NBUF + [
            pltpu.VMEM_SHARED((_N, _D_HALF), jnp.float32),
            pltpu.SemaphoreType.DMA((_NBUF,)),
            pltpu.SemaphoreType.DMA((_NBUF,)),
        ],
    )
    def k(combo_h, vals_h, xa_h, xb_h, oa_h, ob_h,
          idx_t, val_t, r0_v, r1_v, r2_v, acc, gsem, ssem):
        c = lax.axis_index("c")
        s = lax.axis_index("s")
        rows = (r0_v, r1_v, r2_v)

        zeros16 = jnp.zeros((16,), jnp.float32)

        def zrow(i, _):
            for j in range(_D_HALF // 16):
                rows[0][i, pl.ds(j * 16, 16)] = zeros16
            return _

        lax.fori_loop(0, _K, zrow, None)

        def zinit(i, _):
            blk = s + _TILES * i

            @pl.when(blk < _NBLK)
            def _z():
                base = blk * _BLK
                pltpu.sync_copy(rows[0], acc.at[pl.ds(base, _K)])
                pltpu.sync_copy(rows[0].at[pl.ds(0, _BLK - _K)],
                                acc.at[pl.ds(base + _K, _BLK - _K)])
            return _

        lax.fori_loop(0, _MAXB, zinit, None)
        plsc.subcore_barrier()

        def load_idx(ci, b):
            pltpu.sync_copy(combo_h.at[s, ci], idx_t.at[b])
            pltpu.sync_copy(vals_h.at[s, ci], val_t.at[b])

        def gather(b):
            @pl.when(c == 0)
            def _g0():
                pltpu.async_copy(xa_h.at[idx_t.at[b, 0]], rows[b], gsem.at[b])

            @pl.when(c == 1)
            def _g1():
                pltpu.async_copy(xb_h.at[idx_t.at[b, 0]], rows[b], gsem.at[b])

        def gather_wait(b):
            @pl.when(c == 0)
            def _w0():
                pltpu.make_async_copy(
                    xa_h.at[idx_t.at[b, 0]], rows[b], gsem.at[b]).wait()

            @pl.when(c == 1)
            def _w1():
                pltpu.make_async_copy(
                    xb_h.at[idx_t.at[b, 0]], rows[b], gsem.at[b]).wait()

        def scatter_desc(b):
            return pltpu.make_async_copy(
                rows[b], acc.at[idx_t.at[b, 1]], ssem.at[b])

        def scale(b):
            def edge_group(g, _):
                vv = val_t[b, pl.ds(g * 16, 16)]
                for l in range(16):
                    v = vv[l]
                    e = g * 16 + l
                    for j in range(_D_HALF // 16):
                        sl = pl.ds(j * 16, 16)
                        rows[b][e, sl] = rows[b][e, sl] * v
                return _

            lax.fori_loop(0, _K // 16, edge_group, None)

        load_idx(0, 0)
        gather(0)
        load_idx(1, 1)
        gather(1)

        def triple(i, _):
            for b in range(_NBUF):
                ci = _NBUF * i + b
                nb = (b + 2) % _NBUF

                gather_wait(b)

                @pl.when(ci >= 1)
                def _ws():
                    scatter_desc(nb).wait()

                @pl.when(ci + 2 < _CHUNKS)
                def _gi():
                    load_idx(ci + 2, nb)
                    gather(nb)

                scale(b)
                scatter_desc(b).start(add=True)
            return _

        lax.fori_loop(0, (_CHUNKS - 2) // _NBUF, triple, None)
        for ci in (_CHUNKS - 2, _CHUNKS - 1):
            b = ci % _NBUF
            gather_wait(b)
            scale(b)
            scatter_desc(b).start(add=True)
        for b in range(_NBUF):
            scatter_desc(b).wait()
        plsc.subcore_barrier()

        def epi(i, _):
            blk = s + _TILES * i

            @pl.when(blk < _NBLK)
            def _e():
                sl = pl.ds(blk * _BLK, _BLK)

                @pl.when(c == 0)
                def _w0():
                    pltpu.sync_copy(acc.at[sl], oa_h.at[sl])

                @pl.when(c == 1)
                def _w1():
                    pltpu.sync_copy(acc.at[sl], ob_h.at[sl])
            return _

        lax.fori_loop(0, _MAXB, epi, None)

    return k(combo, vals, xa, xb)


def _dense_tc(sa, sb, W1a, W1b, b1r, W2a, W2b):
    bm = 1000

    def body(sa_r, sb_r, w1a_r, w1b_r, b1_r, w2a_r, w2b_r, ta_r, tb_r):
        h = jnp.dot(sa_r[...], w1a_r[...], preferred_element_type=jnp.float32)
        h = h + jnp.dot(sb_r[...], w1b_r[...], preferred_element_type=jnp.float32)
        h = jnp.maximum(h + b1_r[...], 0.0)
        ta_r[...] = jnp.dot(h, w2a_r[...], preferred_element_type=jnp.float32)
        tb_r[...] = jnp.dot(h, w2b_r[...], preferred_element_type=jnp.float32)

    hid = W1a.shape[1]
    return pl.pallas_call(
        body,
        grid=(_N // bm,),
        in_specs=[
            pl.BlockSpec((bm, _D_HALF), lambda i: (i, 0)),
            pl.BlockSpec((bm, _D_HALF), lambda i: (i, 0)),
            pl.BlockSpec((_D_HALF, hid), lambda i: (0, 0)),
            pl.BlockSpec((_D_HALF, hid), lambda i: (0, 0)),
            pl.BlockSpec((1, hid), lambda i: (0, 0)),
            pl.BlockSpec((hid, _D_HALF), lambda i: (0, 0)),
            pl.BlockSpec((hid, _D_HALF), lambda i: (0, 0)),
        ],
        out_specs=[
            pl.BlockSpec((bm, _D_HALF), lambda i: (i, 0)),
            pl.BlockSpec((bm, _D_HALF), lambda i: (i, 0)),
        ],
        out_shape=[
            jax.ShapeDtypeStruct((_N, _D_HALF), jnp.float32),
            jax.ShapeDtypeStruct((_N, _D_HALF), jnp.float32),
        ],
    )(sa, sb, W1a, W1b, b1r, W2a, W2b)


def kernel(x, adj_vals, edge_index, W1, b1, W2, b2):
    src = edge_index[0].astype(jnp.int32)
    dst = edge_index[1].astype(jnp.int32)
    pad = _EPAD - _E
    shape3 = (_TILES, _CHUNKS, _K)
    src = jnp.concatenate([src, jnp.zeros((pad,), jnp.int32)]).reshape(shape3)
    dst = jnp.concatenate([dst, jnp.zeros((pad,), jnp.int32)]).reshape(shape3)
    vals = jnp.concatenate(
        [adj_vals, jnp.zeros((pad,), jnp.float32)]).reshape(shape3)
    combo = jnp.stack([src, dst], axis=2)

    xa = x[:, :_D_HALF]
    xb = x[:, _D_HALF:]
    sa, sb = _spmm_sc(combo, vals, xa, xb)

    ta, tb = _dense_tc(sa, sb, W1[:_D_HALF], W1[_D_HALF:],
                       b1.reshape(1, -1), W2[:, :_D_HALF], W2[:, _D_HALF:])

    oa, ob = _spmm_sc(combo, vals, ta, tb)
    return jnp.concatenate([oa, ob], axis=1) + b2
